# baseline (speedup 1.0000x reference)
import jax
import jax.numpy as jnp
from jax import lax
from jax.experimental import pallas as pl
from jax.experimental.pallas import tpu as pltpu
from jax.experimental.pallas import tpu_sc as plsc

N = 10000
E = 160000
D = 256
H = 256
C = 64

NC = 2
NS = 16
CH = 32

EP = 163840
EROWS = EP // CH
NA = 10008
Z0 = 624
ZL = NA - 15 * Z0
OROWS = 624
OLAST = N - 15 * OROWS

_mesh = plsc.VectorSubcoreMesh(core_axis_name="c", subcore_axis_name="s",
                               num_cores=NC, num_subcores=NS)


def _zero_acc(zw_hbm, sh_ref, sid):
  @pl.when(sid < NS - 1)
  def _():
    pltpu.sync_copy(zw_hbm.at[pl.ds(0, Z0)], sh_ref.at[pl.ds(sid * Z0, Z0)])

  @pl.when(sid == NS - 1)
  def _():
    pltpu.sync_copy(zw_hbm, sh_ref.at[pl.ds((NS - 1) * Z0, ZL)])


def _write_out(sh_ref, out_ref, cid, sid):
  @pl.when(sid < NS - 1)
  def _():
    pltpu.sync_copy(sh_ref.at[pl.ds(sid * OROWS, OROWS)],
                    out_ref.at[cid, pl.ds(sid * OROWS, OROWS)])

  @pl.when(sid == NS - 1)
  def _():
    pltpu.sync_copy(sh_ref.at[pl.ds((NS - 1) * OROWS, OLAST)],
                    out_ref.at[cid, pl.ds((NS - 1) * OROWS, OLAST)])


def _run_ring(y_hbm, src_hbm, dst_hbm, src_v, dst_v, rows, acc_sh,
              sg, ss, tile_base, pr, nph):
  nb = 8
  nq = pr // nb

  def phase(p, carry):
    base = tile_base + p * pr
    pltpu.sync_copy(src_hbm.at[pl.ds(base, pr)], src_v)
    pltpu.sync_copy(dst_hbm.at[pl.ds(base, pr)], dst_v)
    for b in range(nb):
      pltpu.async_copy(y_hbm.at[src_v.at[b]], rows[b], sg[b])

    def quad(j, c):
      i0 = nb * j
      for b in range(nb):
        pltpu.make_async_copy(y_hbm.at[src_v.at[i0 + b]], rows[b],
                              sg[b]).wait()
        pltpu.async_copy(rows[b], acc_sh.at[dst_v.at[i0 + b]], ss[b],
                         add=True)

      @pl.when(j < nq - 1)
      def _():
        for b in range(nb):
          pltpu.make_async_copy(rows[b], acc_sh.at[dst_v.at[i0 + b]],
                                ss[b]).wait()
          pltpu.async_copy(y_hbm.at[src_v.at[i0 + nb + b]], rows[b], sg[b])
      return c
    lax.fori_loop(0, nq, quad, carry)
    for b in range(nb):
      pltpu.make_async_copy(rows[b], acc_sh.at[dst_v.at[pr - nb + b]],
                            ss[b]).wait()
    return carry
  lax.fori_loop(0, nph, phase, 0)


def _deg_edgesplit(dst2d, zerosW, onesW):
  rt = EROWS // (NC * NS)

  def body(dst_hbm, zw_hbm, one_hbm,
           out_hbm,
           dst_v, rows_v, acc_sh, sem):
    cid = lax.axis_index("c")
    sid = lax.axis_index("s")
    base = (cid * NS + sid) * rt

    pltpu.sync_copy(one_hbm, rows_v)
    pltpu.sync_copy(dst_hbm.at[pl.ds(base, rt)], dst_v)
    _zero_acc(zw_hbm, acc_sh, sid)
    plsc.subcore_barrier()

    def chunk(i, carry):
      pltpu.async_copy(rows_v, acc_sh.at[dst_v.at[i]], sem, add=True)
      return carry
    lax.fori_loop(0, rt, chunk, 0)

    def drain(i, carry):
      pltpu.make_async_copy(rows_v, acc_sh.at[dst_v.at[0]], sem).wait()
      return carry
    lax.fori_loop(0, rt, drain, 0)

    plsc.subcore_barrier()
    _write_out(acc_sh, out_hbm, cid, sid)

  f = pl.kernel(
      body,
      out_type=jax.ShapeDtypeStruct((NC, N, 128), jnp.float32),
      mesh=_mesh,
      scratch_types=[
          pltpu.VMEM((rt, CH), jnp.int32),
          pltpu.VMEM((CH, 128), jnp.float32),
          pltpu.VMEM_SHARED((NA, 128), jnp.float32),
          pltpu.SemaphoreType.DMA,
      ],
  )
  return f(dst2d, zerosW, onesW)


def _seg_colsplit_nodeg(y0, y1, src2d, dst2d, zerosW):
  pr = 16
  nph = EROWS // NS // pr

  def body(y0_hbm, y1_hbm, src_hbm, dst_hbm, zw_hbm,
           out_hbm,
           src_v, dst_v, r0, r1, r2, r3, r4, r5, r6, r7, acc_sh,
           sg0, sg1, sg2, sg3, sg4, sg5, sg6, sg7,
           ss0, ss1, ss2, ss3, ss4, ss5, ss6, ss7):
    cid = lax.axis_index("c")
    sid = lax.axis_index("s")
    rows = (r0, r1, r2, r3, r4, r5, r6, r7)
    sg = (sg0, sg1, sg2, sg3, sg4, sg5, sg6, sg7)
    ss = (ss0, ss1, ss2, ss3, ss4, ss5, ss6, ss7)

    _zero_acc(zw_hbm, acc_sh, sid)
    plsc.subcore_barrier()

    tile_base = sid * (pr * nph)

    @pl.when(cid == 0)
    def _():
      _run_ring(y0_hbm, src_hbm, dst_hbm, src_v, dst_v, rows,
                acc_sh, sg, ss, tile_base, pr, nph)

    @pl.when(cid == 1)
    def _():
      _run_ring(y1_hbm, src_hbm, dst_hbm, src_v, dst_v, rows,
                acc_sh, sg, ss, tile_base, pr, nph)

    plsc.subcore_barrier()
    _write_out(acc_sh, out_hbm, cid, sid)

  f = pl.kernel(
      body,
      out_type=jax.ShapeDtypeStruct((NC, N, 128), jnp.float32),
      mesh=_mesh,
      scratch_types=[
          pltpu.VMEM((pr, CH), jnp.int32),
          pltpu.VMEM((pr, CH), jnp.int32),
          pltpu.VMEM((CH, 128), jnp.float32),
          pltpu.VMEM((CH, 128), jnp.float32),
          pltpu.VMEM((CH, 128), jnp.float32),
          pltpu.VMEM((CH, 128), jnp.float32),
          pltpu.VMEM((CH, 128), jnp.float32),
          pltpu.VMEM((CH, 128), jnp.float32),
          pltpu.VMEM((CH, 128), jnp.float32),
          pltpu.VMEM((CH, 128), jnp.float32),
          pltpu.VMEM_SHARED((NA, 128), jnp.float32),
          pltpu.SemaphoreType.DMA,
          pltpu.SemaphoreType.DMA,
          pltpu.SemaphoreType.DMA,
          pltpu.SemaphoreType.DMA,
          pltpu.SemaphoreType.DMA,
          pltpu.SemaphoreType.DMA,
          pltpu.SemaphoreType.DMA,
          pltpu.SemaphoreType.DMA,
          pltpu.SemaphoreType.DMA,
          pltpu.SemaphoreType.DMA,
          pltpu.SemaphoreType.DMA,
          pltpu.SemaphoreType.DMA,
          pltpu.SemaphoreType.DMA,
          pltpu.SemaphoreType.DMA,
          pltpu.SemaphoreType.DMA,
          pltpu.SemaphoreType.DMA,
      ],
  )
  return f(y0, y1, src2d, dst2d, zerosW)


def _seg_edgesplit(ya, yb, src2d, dst2d, zerosW):
  pr = 16
  nph = EROWS // (NC * NS) // pr

  def body(ya_hbm, yb_hbm, src_hbm, dst_hbm, zw_hbm,
           out_hbm,
           src_v, dst_v, r0, r1, r2, r3, r4, r5, r6, r7, acc_sh,
           sg0, sg1, sg2, sg3, sg4, sg5, sg6, sg7,
           ss0, ss1, ss2, ss3, ss4, ss5, ss6, ss7):
    cid = lax.axis_index("c")
    sid = lax.axis_index("s")
    tile_base = (cid * NS + sid) * (pr * nph)
    rows = (r0, r1, r2, r3, r4, r5, r6, r7)
    sg = (sg0, sg1, sg2, sg3, sg4, sg5, sg6, sg7)
    ss = (ss0, ss1, ss2, ss3, ss4, ss5, ss6, ss7)

    _zero_acc(zw_hbm, acc_sh, sid)
    plsc.subcore_barrier()

    @pl.when(cid == 0)
    def _():
      _run_ring(ya_hbm, src_hbm, dst_hbm, src_v, dst_v, rows,
                acc_sh, sg, ss, tile_base, pr, nph)

    @pl.when(cid == 1)
    def _():
      _run_ring(yb_hbm, src_hbm, dst_hbm, src_v, dst_v, rows,
                acc_sh, sg, ss, tile_base, pr, nph)

    plsc.subcore_barrier()
    _write_out(acc_sh, out_hbm, cid, sid)

  f = pl.kernel(
      body,
      out_type=jax.ShapeDtypeStruct((NC, N, 128), jnp.float32),
      mesh=_mesh,
      scratch_types=[
          pltpu.VMEM((pr, CH), jnp.int32),
          pltpu.VMEM((pr, CH), jnp.int32),
          pltpu.VMEM((CH, 128), jnp.float32),
          pltpu.VMEM((CH, 128), jnp.float32),
          pltpu.VMEM((CH, 128), jnp.float32),
          pltpu.VMEM((CH, 128), jnp.float32),
          pltpu.VMEM((CH, 128), jnp.float32),
          pltpu.VMEM((CH, 128), jnp.float32),
          pltpu.VMEM((CH, 128), jnp.float32),
          pltpu.VMEM((CH, 128), jnp.float32),
          pltpu.VMEM_SHARED((NA, 128), jnp.float32),
          pltpu.SemaphoreType.DMA,
          pltpu.SemaphoreType.DMA,
          pltpu.SemaphoreType.DMA,
          pltpu.SemaphoreType.DMA,
          pltpu.SemaphoreType.DMA,
          pltpu.SemaphoreType.DMA,
          pltpu.SemaphoreType.DMA,
          pltpu.SemaphoreType.DMA,
          pltpu.SemaphoreType.DMA,
          pltpu.SemaphoreType.DMA,
          pltpu.SemaphoreType.DMA,
          pltpu.SemaphoreType.DMA,
          pltpu.SemaphoreType.DMA,
          pltpu.SemaphoreType.DMA,
          pltpu.SemaphoreType.DMA,
          pltpu.SemaphoreType.DMA,
      ],
  )
  return f(ya, yb, src2d, dst2d, zerosW)


_BM = 2000


def _tc_first(x, W_self, W_neigh):
  def body(x_ref, ws_ref, wn_ref, s_ref, y0_ref, y1_ref):
    xb = x_ref[...]
    s_ref[...] = jnp.dot(xb, ws_ref[...], preferred_element_type=jnp.float32)
    y = jnp.dot(xb, wn_ref[...], preferred_element_type=jnp.float32)
    y0_ref[...] = y[:, :128]
    y1_ref[...] = y[:, 128:]

  return pl.pallas_call(
      body,
      grid=(N // _BM,),
      in_specs=[
          pl.BlockSpec((_BM, D), lambda i: (i, 0)),
          pl.BlockSpec((D, H), lambda i: (0, 0)),
          pl.BlockSpec((D, H), lambda i: (0, 0)),
      ],
      out_specs=[
          pl.BlockSpec((_BM, H), lambda i: (i, 0)),
          pl.BlockSpec((_BM, 128), lambda i: (i, 0)),
          pl.BlockSpec((_BM, 128), lambda i: (i, 0)),
      ],
      out_shape=[
          jax.ShapeDtypeStruct((N, H), jnp.float32),
          jax.ShapeDtypeStruct((N, 128), jnp.float32),
          jax.ShapeDtypeStruct((N, 128), jnp.float32),
      ],
  )(x, W_self, W_neigh)


def _tc_mid(s_prev, agg, deg, b, W_self, W_neigh, n_out, split):
  def body(sp_ref, a_ref, d_ref, b_ref, ws_ref, wn_ref, *outs):
    d = d_ref[0, :, 0:1] + d_ref[1, :, 0:1]
    inv = 1.0 / jnp.maximum(d, 1.0)
    aggf = jnp.concatenate([a_ref[0], a_ref[1]], axis=-1)
    h = jax.nn.relu(sp_ref[...] + aggf * inv + b_ref[0])
    s = jnp.dot(h, ws_ref[...], preferred_element_type=jnp.float32)
    y = jnp.dot(h, wn_ref[...], preferred_element_type=jnp.float32)
    outs[0][...] = s
    if split:
      outs[1][...] = y[:, :128]
      outs[2][...] = y[:, 128:]
    else:
      pad = jnp.zeros((y.shape[0], 128 - n_out), jnp.float32)
      yp = jnp.concatenate([y, pad], axis=1)
      outs[1][...] = yp
      outs[2][...] = yp

  if split:
    out_specs = [
        pl.BlockSpec((_BM, n_out), lambda i: (i, 0)),
        pl.BlockSpec((_BM, 128), lambda i: (i, 0)),
        pl.BlockSpec((_BM, 128), lambda i: (i, 0)),
    ]
    out_shape = [
        jax.ShapeDtypeStruct((N, n_out), jnp.float32),
        jax.ShapeDtypeStruct((N, 128), jnp.float32),
        jax.ShapeDtypeStruct((N, 128), jnp.float32),
    ]
  else:
    out_specs = [
        pl.BlockSpec((_BM, n_out), lambda i: (i, 0)),
        pl.BlockSpec((_BM, 128), lambda i: (i, 0)),
        pl.BlockSpec((_BM, 128), lambda i: (i, 0)),
    ]
    out_shape = [
        jax.ShapeDtypeStruct((N, n_out), jnp.float32),
        jax.ShapeDtypeStruct((N, 128), jnp.float32),
        jax.ShapeDtypeStruct((N, 128), jnp.float32),
    ]

  return pl.pallas_call(
      body,
      grid=(N // _BM,),
      in_specs=[
          pl.BlockSpec((_BM, H), lambda i: (i, 0)),
          pl.BlockSpec((NC, _BM, 128), lambda i: (0, i, 0)),
          pl.BlockSpec((NC, _BM, 128), lambda i: (0, i, 0)),
          pl.BlockSpec((1, H), lambda i: (0, 0)),
          pl.BlockSpec((H, n_out), lambda i: (0, 0)),
          pl.BlockSpec((H, n_out), lambda i: (0, 0)),
      ],
      out_specs=out_specs,
      out_shape=out_shape,
  )(s_prev, agg, deg, b, W_self, W_neigh)


def _tc_final(s3, agg3, deg, b):
  def body(sp_ref, a_ref, d_ref, b_ref, o_ref):
    d = d_ref[0, :, 0:1] + d_ref[1, :, 0:1]
    inv = 1.0 / jnp.maximum(d, 1.0)
    aggf = a_ref[0, :, :C] + a_ref[1, :, :C]
    o_ref[...] = sp_ref[...] + aggf * inv + b_ref[0]

  return pl.pallas_call(
      body,
      grid=(N // _BM,),
      in_specs=[
          pl.BlockSpec((_BM, C), lambda i: (i, 0)),
          pl.BlockSpec((NC, _BM, 128), lambda i: (0, i, 0)),
          pl.BlockSpec((NC, _BM, 128), lambda i: (0, i, 0)),
          pl.BlockSpec((1, C), lambda i: (0, 0)),
      ],
      out_specs=pl.BlockSpec((_BM, C), lambda i: (i, 0)),
      out_shape=jax.ShapeDtypeStruct((N, C), jnp.float32),
  )(s3, agg3, deg, b)


def kernel(x, edge_index, W_self1, W_neigh1, b1, W_self2, W_neigh2, b2,
           W_self3, W_neigh3, b3):
  src = edge_index[0].astype(jnp.int32)
  dst = edge_index[1].astype(jnp.int32)
  pad = EP - E
  src2d = jnp.concatenate(
      [src, jnp.zeros((pad,), jnp.int32)]).reshape(EROWS, CH)
  dst2d = jnp.concatenate(
      [dst, jnp.full((pad,), N, jnp.int32)]).reshape(EROWS, CH)

  zeros128 = jnp.zeros((ZL, 128), jnp.float32)
  ones128 = jnp.ones((CH, 128), jnp.float32)

  b1r = b1.reshape(1, H)
  b2r = b2.reshape(1, H)
  b3r = b3.reshape(1, C)

  s1, y10, y11 = _tc_first(x, W_self1, W_neigh1)
  agg1 = _seg_colsplit_nodeg(y10, y11, src2d, dst2d, zeros128)
  deg = _deg_edgesplit(dst2d, zeros128, ones128)
  s2, y20, y21 = _tc_mid(s1, agg1, deg, b1r, W_self2, W_neigh2, H, True)
  agg2 = _seg_colsplit_nodeg(y20, y21, src2d, dst2d, zeros128)
  s3, y3a, y3b = _tc_mid(s2, agg2, deg, b2r, W_self3, W_neigh3, C, False)
  agg3 = _seg_edgesplit(y3a, y3b, src2d, dst2d, zeros128)
  return _tc_final(s3, agg3, deg, b3r)


# speedup vs baseline: 1.2438x; 1.2438x over previous
import jax
import jax.numpy as jnp
from jax import lax
from jax.experimental import pallas as pl
from jax.experimental.pallas import tpu as pltpu
from jax.experimental.pallas import tpu_sc as plsc

N = 10000
E = 160000
D = 256
H = 256
C = 64

NC = 2
NS = 16
CH = 64

EP = 163840
EROWS = EP // CH
NA = 10008
Z0 = 624
ZL = NA - 15 * Z0
OROWS = 624
OLAST = N - 15 * OROWS

_mesh = plsc.VectorSubcoreMesh(core_axis_name="c", subcore_axis_name="s",
                               num_cores=NC, num_subcores=NS)


def _zero_acc(zw_hbm, sh_ref, sid):
  @pl.when(sid < NS - 1)
  def _():
    pltpu.sync_copy(zw_hbm.at[pl.ds(0, Z0)], sh_ref.at[pl.ds(sid * Z0, Z0)])

  @pl.when(sid == NS - 1)
  def _():
    pltpu.sync_copy(zw_hbm, sh_ref.at[pl.ds((NS - 1) * Z0, ZL)])


def _write_out(sh_ref, out_ref, cid, sid):
  @pl.when(sid < NS - 1)
  def _():
    pltpu.sync_copy(sh_ref.at[pl.ds(sid * OROWS, OROWS)],
                    out_ref.at[cid, pl.ds(sid * OROWS, OROWS)])

  @pl.when(sid == NS - 1)
  def _():
    pltpu.sync_copy(sh_ref.at[pl.ds((NS - 1) * OROWS, OLAST)],
                    out_ref.at[cid, pl.ds((NS - 1) * OROWS, OLAST)])


def _run_ring(y_hbm, src_hbm, dst_hbm, src_v, dst_v, rows, acc_sh,
              sg, ss, tile_base, pr, nph):
  nb = 4
  nq = pr // nb

  def phase(p, carry):
    base = tile_base + p * pr
    pltpu.sync_copy(src_hbm.at[pl.ds(base, pr)], src_v)
    pltpu.sync_copy(dst_hbm.at[pl.ds(base, pr)], dst_v)
    for b in range(nb):
      pltpu.async_copy(y_hbm.at[src_v.at[b]], rows[b], sg[b])

    def quad(j, c):
      i0 = nb * j
      for b in range(nb):
        pltpu.make_async_copy(y_hbm.at[src_v.at[i0 + b]], rows[b],
                              sg[b]).wait()
        pltpu.async_copy(rows[b], acc_sh.at[dst_v.at[i0 + b]], ss[b],
                         add=True)

      @pl.when(j < nq - 1)
      def _():
        for b in range(nb):
          pltpu.make_async_copy(rows[b], acc_sh.at[dst_v.at[i0 + b]],
                                ss[b]).wait()
          pltpu.async_copy(y_hbm.at[src_v.at[i0 + nb + b]], rows[b], sg[b])
      return c
    lax.fori_loop(0, nq, quad, carry)
    for b in range(nb):
      pltpu.make_async_copy(rows[b], acc_sh.at[dst_v.at[pr - nb + b]],
                            ss[b]).wait()
    return carry
  lax.fori_loop(0, nph, phase, 0)


def _deg_edgesplit(dst2d, zerosW, onesW):
  rt = EROWS // (NC * NS)

  def body(dst_hbm, zw_hbm, one_hbm,
           out_hbm,
           dst_v, rows_v, acc_sh, sem):
    cid = lax.axis_index("c")
    sid = lax.axis_index("s")
    base = (cid * NS + sid) * rt

    pltpu.sync_copy(one_hbm, rows_v)
    pltpu.sync_copy(dst_hbm.at[pl.ds(base, rt)], dst_v)
    _zero_acc(zw_hbm, acc_sh, sid)
    plsc.subcore_barrier()

    def chunk(i, carry):
      pltpu.async_copy(rows_v, acc_sh.at[dst_v.at[i]], sem, add=True)
      return carry
    lax.fori_loop(0, rt, chunk, 0)

    def drain(i, carry):
      pltpu.make_async_copy(rows_v, acc_sh.at[dst_v.at[0]], sem).wait()
      return carry
    lax.fori_loop(0, rt, drain, 0)

    plsc.subcore_barrier()
    _write_out(acc_sh, out_hbm, cid, sid)

  f = pl.kernel(
      body,
      out_type=jax.ShapeDtypeStruct((NC, N, 128), jnp.float32),
      mesh=_mesh,
      scratch_types=[
          pltpu.VMEM((rt, CH), jnp.int32),
          pltpu.VMEM((CH, 128), jnp.float32),
          pltpu.VMEM_SHARED((NA, 128), jnp.float32),
          pltpu.SemaphoreType.DMA,
      ],
  )
  return f(dst2d, zerosW, onesW)


def _seg_colsplit_nodeg(y0, y1, src2d, dst2d, zerosW):
  pr = 16
  nph = EROWS // NS // pr

  def body(y0_hbm, y1_hbm, src_hbm, dst_hbm, zw_hbm,
           out_hbm,
           src_v, dst_v, r0, r1, r2, r3, acc_sh,
           sg0, sg1, sg2, sg3, ss0, ss1, ss2, ss3):
    cid = lax.axis_index("c")
    sid = lax.axis_index("s")
    rows = (r0, r1, r2, r3)
    sg = (sg0, sg1, sg2, sg3)
    ss = (ss0, ss1, ss2, ss3)

    _zero_acc(zw_hbm, acc_sh, sid)
    plsc.subcore_barrier()

    tile_base = sid * (pr * nph)

    @pl.when(cid == 0)
    def _():
      _run_ring(y0_hbm, src_hbm, dst_hbm, src_v, dst_v, rows,
                acc_sh, sg, ss, tile_base, pr, nph)

    @pl.when(cid == 1)
    def _():
      _run_ring(y1_hbm, src_hbm, dst_hbm, src_v, dst_v, rows,
                acc_sh, sg, ss, tile_base, pr, nph)

    plsc.subcore_barrier()
    _write_out(acc_sh, out_hbm, cid, sid)

  f = pl.kernel(
      body,
      out_type=jax.ShapeDtypeStruct((NC, N, 128), jnp.float32),
      mesh=_mesh,
      scratch_types=[
          pltpu.VMEM((pr, CH), jnp.int32),
          pltpu.VMEM((pr, CH), jnp.int32),
          pltpu.VMEM((CH, 128), jnp.float32),
          pltpu.VMEM((CH, 128), jnp.float32),
          pltpu.VMEM((CH, 128), jnp.float32),
          pltpu.VMEM((CH, 128), jnp.float32),
          pltpu.VMEM_SHARED((NA, 128), jnp.float32),
          pltpu.SemaphoreType.DMA,
          pltpu.SemaphoreType.DMA,
          pltpu.SemaphoreType.DMA,
          pltpu.SemaphoreType.DMA,
          pltpu.SemaphoreType.DMA,
          pltpu.SemaphoreType.DMA,
          pltpu.SemaphoreType.DMA,
          pltpu.SemaphoreType.DMA,
      ],
  )
  return f(y0, y1, src2d, dst2d, zerosW)


def _seg_edgesplit(ya, yb, src2d, dst2d, zerosW):
  pr = 16
  nph = EROWS // (NC * NS) // pr

  def body(ya_hbm, yb_hbm, src_hbm, dst_hbm, zw_hbm,
           out_hbm,
           src_v, dst_v, r0, r1, r2, r3, acc_sh,
           sg0, sg1, sg2, sg3, ss0, ss1, ss2, ss3):
    cid = lax.axis_index("c")
    sid = lax.axis_index("s")
    tile_base = (cid * NS + sid) * (pr * nph)
    rows = (r0, r1, r2, r3)
    sg = (sg0, sg1, sg2, sg3)
    ss = (ss0, ss1, ss2, ss3)

    _zero_acc(zw_hbm, acc_sh, sid)
    plsc.subcore_barrier()

    @pl.when(cid == 0)
    def _():
      _run_ring(ya_hbm, src_hbm, dst_hbm, src_v, dst_v, rows,
                acc_sh, sg, ss, tile_base, pr, nph)

    @pl.when(cid == 1)
    def _():
      _run_ring(yb_hbm, src_hbm, dst_hbm, src_v, dst_v, rows,
                acc_sh, sg, ss, tile_base, pr, nph)

    plsc.subcore_barrier()
    _write_out(acc_sh, out_hbm, cid, sid)

  f = pl.kernel(
      body,
      out_type=jax.ShapeDtypeStruct((NC, N, 128), jnp.float32),
      mesh=_mesh,
      scratch_types=[
          pltpu.VMEM((pr, CH), jnp.int32),
          pltpu.VMEM((pr, CH), jnp.int32),
          pltpu.VMEM((CH, 128), jnp.float32),
          pltpu.VMEM((CH, 128), jnp.float32),
          pltpu.VMEM((CH, 128), jnp.float32),
          pltpu.VMEM((CH, 128), jnp.float32),
          pltpu.VMEM_SHARED((NA, 128), jnp.float32),
          pltpu.SemaphoreType.DMA,
          pltpu.SemaphoreType.DMA,
          pltpu.SemaphoreType.DMA,
          pltpu.SemaphoreType.DMA,
          pltpu.SemaphoreType.DMA,
          pltpu.SemaphoreType.DMA,
          pltpu.SemaphoreType.DMA,
          pltpu.SemaphoreType.DMA,
      ],
  )
  return f(ya, yb, src2d, dst2d, zerosW)


_BM = 2000


def _tc_first(x, W_self, W_neigh):
  def body(x_ref, ws_ref, wn_ref, s_ref, y0_ref, y1_ref):
    xb = x_ref[...]
    s_ref[...] = jnp.dot(xb, ws_ref[...], preferred_element_type=jnp.float32)
    y = jnp.dot(xb, wn_ref[...], preferred_element_type=jnp.float32)
    y0_ref[...] = y[:, :128]
    y1_ref[...] = y[:, 128:]

  return pl.pallas_call(
      body,
      grid=(N // _BM,),
      in_specs=[
          pl.BlockSpec((_BM, D), lambda i: (i, 0)),
          pl.BlockSpec((D, H), lambda i: (0, 0)),
          pl.BlockSpec((D, H), lambda i: (0, 0)),
      ],
      out_specs=[
          pl.BlockSpec((_BM, H), lambda i: (i, 0)),
          pl.BlockSpec((_BM, 128), lambda i: (i, 0)),
          pl.BlockSpec((_BM, 128), lambda i: (i, 0)),
      ],
      out_shape=[
          jax.ShapeDtypeStruct((N, H), jnp.float32),
          jax.ShapeDtypeStruct((N, 128), jnp.float32),
          jax.ShapeDtypeStruct((N, 128), jnp.float32),
      ],
  )(x, W_self, W_neigh)


def _tc_mid(s_prev, agg, deg, b, W_self, W_neigh, n_out, split):
  def body(sp_ref, a_ref, d_ref, b_ref, ws_ref, wn_ref, *outs):
    d = d_ref[0, :, 0:1] + d_ref[1, :, 0:1]
    inv = 1.0 / jnp.maximum(d, 1.0)
    aggf = jnp.concatenate([a_ref[0], a_ref[1]], axis=-1)
    h = jax.nn.relu(sp_ref[...] + aggf * inv + b_ref[0])
    s = jnp.dot(h, ws_ref[...], preferred_element_type=jnp.float32)
    y = jnp.dot(h, wn_ref[...], preferred_element_type=jnp.float32)
    outs[0][...] = s
    if split:
      outs[1][...] = y[:, :128]
      outs[2][...] = y[:, 128:]
    else:
      pad = jnp.zeros((y.shape[0], 128 - n_out), jnp.float32)
      yp = jnp.concatenate([y, pad], axis=1)
      outs[1][...] = yp
      outs[2][...] = yp

  if split:
    out_specs = [
        pl.BlockSpec((_BM, n_out), lambda i: (i, 0)),
        pl.BlockSpec((_BM, 128), lambda i: (i, 0)),
        pl.BlockSpec((_BM, 128), lambda i: (i, 0)),
    ]
    out_shape = [
        jax.ShapeDtypeStruct((N, n_out), jnp.float32),
        jax.ShapeDtypeStruct((N, 128), jnp.float32),
        jax.ShapeDtypeStruct((N, 128), jnp.float32),
    ]
  else:
    out_specs = [
        pl.BlockSpec((_BM, n_out), lambda i: (i, 0)),
        pl.BlockSpec((_BM, 128), lambda i: (i, 0)),
        pl.BlockSpec((_BM, 128), lambda i: (i, 0)),
    ]
    out_shape = [
        jax.ShapeDtypeStruct((N, n_out), jnp.float32),
        jax.ShapeDtypeStruct((N, 128), jnp.float32),
        jax.ShapeDtypeStruct((N, 128), jnp.float32),
    ]

  return pl.pallas_call(
      body,
      grid=(N // _BM,),
      in_specs=[
          pl.BlockSpec((_BM, H), lambda i: (i, 0)),
          pl.BlockSpec((NC, _BM, 128), lambda i: (0, i, 0)),
          pl.BlockSpec((NC, _BM, 128), lambda i: (0, i, 0)),
          pl.BlockSpec((1, H), lambda i: (0, 0)),
          pl.BlockSpec((H, n_out), lambda i: (0, 0)),
          pl.BlockSpec((H, n_out), lambda i: (0, 0)),
      ],
      out_specs=out_specs,
      out_shape=out_shape,
  )(s_prev, agg, deg, b, W_self, W_neigh)


def _tc_final(s3, agg3, deg, b):
  def body(sp_ref, a_ref, d_ref, b_ref, o_ref):
    d = d_ref[0, :, 0:1] + d_ref[1, :, 0:1]
    inv = 1.0 / jnp.maximum(d, 1.0)
    aggf = a_ref[0, :, :C] + a_ref[1, :, :C]
    o_ref[...] = sp_ref[...] + aggf * inv + b_ref[0]

  return pl.pallas_call(
      body,
      grid=(N // _BM,),
      in_specs=[
          pl.BlockSpec((_BM, C), lambda i: (i, 0)),
          pl.BlockSpec((NC, _BM, 128), lambda i: (0, i, 0)),
          pl.BlockSpec((NC, _BM, 128), lambda i: (0, i, 0)),
          pl.BlockSpec((1, C), lambda i: (0, 0)),
      ],
      out_specs=pl.BlockSpec((_BM, C), lambda i: (i, 0)),
      out_shape=jax.ShapeDtypeStruct((N, C), jnp.float32),
  )(s3, agg3, deg, b)


def kernel(x, edge_index, W_self1, W_neigh1, b1, W_self2, W_neigh2, b2,
           W_self3, W_neigh3, b3):
  src = edge_index[0].astype(jnp.int32)
  dst = edge_index[1].astype(jnp.int32)
  pad = EP - E
  src2d = jnp.concatenate(
      [src, jnp.zeros((pad,), jnp.int32)]).reshape(EROWS, CH)
  dst2d = jnp.concatenate(
      [dst, jnp.full((pad,), N, jnp.int32)]).reshape(EROWS, CH)

  zeros128 = jnp.zeros((ZL, 128), jnp.float32)
  ones128 = jnp.ones((CH, 128), jnp.float32)

  b1r = b1.reshape(1, H)
  b2r = b2.reshape(1, H)
  b3r = b3.reshape(1, C)

  s1, y10, y11 = _tc_first(x, W_self1, W_neigh1)
  agg1 = _seg_colsplit_nodeg(y10, y11, src2d, dst2d, zeros128)
  deg = _deg_edgesplit(dst2d, zeros128, ones128)
  s2, y20, y21 = _tc_mid(s1, agg1, deg, b1r, W_self2, W_neigh2, H, True)
  agg2 = _seg_colsplit_nodeg(y20, y21, src2d, dst2d, zeros128)
  s3, y3a, y3b = _tc_mid(s2, agg2, deg, b2r, W_self3, W_neigh3, C, False)
  agg3 = _seg_edgesplit(y3a, y3b, src2d, dst2d, zeros128)
  return _tc_final(s3, agg3, deg, b3r)


# speedup vs baseline: 1.2827x; 1.0313x over previous
import jax
import jax.numpy as jnp
from jax import lax
from jax.experimental import pallas as pl
from jax.experimental.pallas import tpu as pltpu
from jax.experimental.pallas import tpu_sc as plsc

N = 10000
E = 160000
D = 256
H = 256
C = 64

NC = 2
NS = 16
CH = 64

EP = 163840
EROWS = EP // CH
NA = 10008
Z0 = 624
ZL = NA - 15 * Z0
OROWS = 624
OLAST = N - 15 * OROWS

_mesh = plsc.VectorSubcoreMesh(core_axis_name="c", subcore_axis_name="s",
                               num_cores=NC, num_subcores=NS)


def _zero_acc(zw_hbm, sh_ref, sid):
  @pl.when(sid < NS - 1)
  def _():
    pltpu.sync_copy(zw_hbm.at[pl.ds(0, Z0)], sh_ref.at[pl.ds(sid * Z0, Z0)])

  @pl.when(sid == NS - 1)
  def _():
    pltpu.sync_copy(zw_hbm, sh_ref.at[pl.ds((NS - 1) * Z0, ZL)])


def _write_out(sh_ref, out_ref, cid, sid):
  @pl.when(sid < NS - 1)
  def _():
    pltpu.sync_copy(sh_ref.at[pl.ds(sid * OROWS, OROWS)],
                    out_ref.at[cid, pl.ds(sid * OROWS, OROWS)])

  @pl.when(sid == NS - 1)
  def _():
    pltpu.sync_copy(sh_ref.at[pl.ds((NS - 1) * OROWS, OLAST)],
                    out_ref.at[cid, pl.ds((NS - 1) * OROWS, OLAST)])


def _run_ring(y_hbm, src_hbm, dst_hbm, src_v, dst_v, rows, acc_sh,
              sg, ss, tile_base, pr, nph):
  nb = 4
  nq = pr // nb

  def phase(p, carry):
    base = tile_base + p * pr
    pltpu.sync_copy(src_hbm.at[pl.ds(base, pr)], src_v)
    pltpu.sync_copy(dst_hbm.at[pl.ds(base, pr)], dst_v)
    for b in range(nb):
      pltpu.async_copy(y_hbm.at[src_v.at[b]], rows[b], sg[b])

    def quad(j, c):
      i0 = nb * j
      for b in range(nb):
        pltpu.make_async_copy(y_hbm.at[src_v.at[i0 + b]], rows[b],
                              sg[b]).wait()
        pltpu.async_copy(rows[b], acc_sh.at[dst_v.at[i0 + b]], ss[b],
                         add=True)

      @pl.when(j < nq - 1)
      def _():
        for b in range(nb):
          pltpu.make_async_copy(rows[b], acc_sh.at[dst_v.at[i0 + b]],
                                ss[b]).wait()
          pltpu.async_copy(y_hbm.at[src_v.at[i0 + nb + b]], rows[b], sg[b])
      return c
    lax.fori_loop(0, nq, quad, carry)
    for b in range(nb):
      pltpu.make_async_copy(rows[b], acc_sh.at[dst_v.at[pr - nb + b]],
                            ss[b]).wait()
    return carry
  lax.fori_loop(0, nph, phase, 0)


def _deg_edgesplit(dst2d, zerosW, onesW):
  rt = EROWS // (NC * NS)

  def body(dst_hbm, zw_hbm, one_hbm,
           out_hbm,
           dst_v, rows_v, acc_sh, sem):
    cid = lax.axis_index("c")
    sid = lax.axis_index("s")
    base = (cid * NS + sid) * rt

    pltpu.sync_copy(one_hbm, rows_v)
    pltpu.sync_copy(dst_hbm.at[pl.ds(base, rt)], dst_v)
    _zero_acc(zw_hbm, acc_sh, sid)
    plsc.subcore_barrier()

    def chunk(i, carry):
      pltpu.async_copy(rows_v, acc_sh.at[dst_v.at[i]], sem, add=True)
      return carry
    lax.fori_loop(0, rt, chunk, 0)

    def drain(i, carry):
      pltpu.make_async_copy(rows_v, acc_sh.at[dst_v.at[0]], sem).wait()
      return carry
    lax.fori_loop(0, rt, drain, 0)

    plsc.subcore_barrier()
    _write_out(acc_sh, out_hbm, cid, sid)

  f = pl.kernel(
      body,
      out_type=jax.ShapeDtypeStruct((NC, N, 128), jnp.float32),
      mesh=_mesh,
      scratch_types=[
          pltpu.VMEM((rt, CH), jnp.int32),
          pltpu.VMEM((CH, 128), jnp.float32),
          pltpu.VMEM_SHARED((NA, 128), jnp.float32),
          pltpu.SemaphoreType.DMA,
      ],
  )
  return f(dst2d, zerosW, onesW)


def _seg_colsplit_nodeg(y0, y1, src2d, dst2d, zerosW):
  pr = 32
  nph = EROWS // NS // pr

  def body(y0_hbm, y1_hbm, src_hbm, dst_hbm, zw_hbm,
           out_hbm,
           src_v, dst_v, r0, r1, r2, r3, acc_sh,
           sg0, sg1, sg2, sg3, ss0, ss1, ss2, ss3):
    cid = lax.axis_index("c")
    sid = lax.axis_index("s")
    rows = (r0, r1, r2, r3)
    sg = (sg0, sg1, sg2, sg3)
    ss = (ss0, ss1, ss2, ss3)

    _zero_acc(zw_hbm, acc_sh, sid)
    plsc.subcore_barrier()

    tile_base = sid * (pr * nph)

    @pl.when(cid == 0)
    def _():
      _run_ring(y0_hbm, src_hbm, dst_hbm, src_v, dst_v, rows,
                acc_sh, sg, ss, tile_base, pr, nph)

    @pl.when(cid == 1)
    def _():
      _run_ring(y1_hbm, src_hbm, dst_hbm, src_v, dst_v, rows,
                acc_sh, sg, ss, tile_base, pr, nph)

    plsc.subcore_barrier()
    _write_out(acc_sh, out_hbm, cid, sid)

  f = pl.kernel(
      body,
      out_type=jax.ShapeDtypeStruct((NC, N, 128), jnp.float32),
      mesh=_mesh,
      scratch_types=[
          pltpu.VMEM((pr, CH), jnp.int32),
          pltpu.VMEM((pr, CH), jnp.int32),
          pltpu.VMEM((CH, 128), jnp.float32),
          pltpu.VMEM((CH, 128), jnp.float32),
          pltpu.VMEM((CH, 128), jnp.float32),
          pltpu.VMEM((CH, 128), jnp.float32),
          pltpu.VMEM_SHARED((NA, 128), jnp.float32),
          pltpu.SemaphoreType.DMA,
          pltpu.SemaphoreType.DMA,
          pltpu.SemaphoreType.DMA,
          pltpu.SemaphoreType.DMA,
          pltpu.SemaphoreType.DMA,
          pltpu.SemaphoreType.DMA,
          pltpu.SemaphoreType.DMA,
          pltpu.SemaphoreType.DMA,
      ],
  )
  return f(y0, y1, src2d, dst2d, zerosW)


def _seg_edgesplit(ya, yb, src2d, dst2d, zerosW):
  pr = 16
  nph = EROWS // (NC * NS) // pr

  def body(ya_hbm, yb_hbm, src_hbm, dst_hbm, zw_hbm,
           out_hbm,
           src_v, dst_v, r0, r1, r2, r3, acc_sh,
           sg0, sg1, sg2, sg3, ss0, ss1, ss2, ss3):
    cid = lax.axis_index("c")
    sid = lax.axis_index("s")
    tile_base = (cid * NS + sid) * (pr * nph)
    rows = (r0, r1, r2, r3)
    sg = (sg0, sg1, sg2, sg3)
    ss = (ss0, ss1, ss2, ss3)

    _zero_acc(zw_hbm, acc_sh, sid)
    plsc.subcore_barrier()

    @pl.when(cid == 0)
    def _():
      _run_ring(ya_hbm, src_hbm, dst_hbm, src_v, dst_v, rows,
                acc_sh, sg, ss, tile_base, pr, nph)

    @pl.when(cid == 1)
    def _():
      _run_ring(yb_hbm, src_hbm, dst_hbm, src_v, dst_v, rows,
                acc_sh, sg, ss, tile_base, pr, nph)

    plsc.subcore_barrier()
    _write_out(acc_sh, out_hbm, cid, sid)

  f = pl.kernel(
      body,
      out_type=jax.ShapeDtypeStruct((NC, N, 128), jnp.float32),
      mesh=_mesh,
      scratch_types=[
          pltpu.VMEM((pr, CH), jnp.int32),
          pltpu.VMEM((pr, CH), jnp.int32),
          pltpu.VMEM((CH, 128), jnp.float32),
          pltpu.VMEM((CH, 128), jnp.float32),
          pltpu.VMEM((CH, 128), jnp.float32),
          pltpu.VMEM((CH, 128), jnp.float32),
          pltpu.VMEM_SHARED((NA, 128), jnp.float32),
          pltpu.SemaphoreType.DMA,
          pltpu.SemaphoreType.DMA,
          pltpu.SemaphoreType.DMA,
          pltpu.SemaphoreType.DMA,
          pltpu.SemaphoreType.DMA,
          pltpu.SemaphoreType.DMA,
          pltpu.SemaphoreType.DMA,
          pltpu.SemaphoreType.DMA,
      ],
  )
  return f(ya, yb, src2d, dst2d, zerosW)


_BM = 2000


def _tc_first(x, W_self, W_neigh):
  def body(x_ref, ws_ref, wn_ref, s_ref, y0_ref, y1_ref):
    xb = x_ref[...]
    s_ref[...] = jnp.dot(xb, ws_ref[...], preferred_element_type=jnp.float32)
    y = jnp.dot(xb, wn_ref[...], preferred_element_type=jnp.float32)
    y0_ref[...] = y[:, :128]
    y1_ref[...] = y[:, 128:]

  return pl.pallas_call(
      body,
      grid=(N // _BM,),
      in_specs=[
          pl.BlockSpec((_BM, D), lambda i: (i, 0)),
          pl.BlockSpec((D, H), lambda i: (0, 0)),
          pl.BlockSpec((D, H), lambda i: (0, 0)),
      ],
      out_specs=[
          pl.BlockSpec((_BM, H), lambda i: (i, 0)),
          pl.BlockSpec((_BM, 128), lambda i: (i, 0)),
          pl.BlockSpec((_BM, 128), lambda i: (i, 0)),
      ],
      out_shape=[
          jax.ShapeDtypeStruct((N, H), jnp.float32),
          jax.ShapeDtypeStruct((N, 128), jnp.float32),
          jax.ShapeDtypeStruct((N, 128), jnp.float32),
      ],
  )(x, W_self, W_neigh)


def _tc_mid(s_prev, agg, deg, b, W_self, W_neigh, n_out, split):
  def body(sp_ref, a_ref, d_ref, b_ref, ws_ref, wn_ref, *outs):
    d = d_ref[0, :, 0:1] + d_ref[1, :, 0:1]
    inv = 1.0 / jnp.maximum(d, 1.0)
    aggf = jnp.concatenate([a_ref[0], a_ref[1]], axis=-1)
    h = jax.nn.relu(sp_ref[...] + aggf * inv + b_ref[0])
    s = jnp.dot(h, ws_ref[...], preferred_element_type=jnp.float32)
    y = jnp.dot(h, wn_ref[...], preferred_element_type=jnp.float32)
    outs[0][...] = s
    if split:
      outs[1][...] = y[:, :128]
      outs[2][...] = y[:, 128:]
    else:
      pad = jnp.zeros((y.shape[0], 128 - n_out), jnp.float32)
      yp = jnp.concatenate([y, pad], axis=1)
      outs[1][...] = yp
      outs[2][...] = yp

  if split:
    out_specs = [
        pl.BlockSpec((_BM, n_out), lambda i: (i, 0)),
        pl.BlockSpec((_BM, 128), lambda i: (i, 0)),
        pl.BlockSpec((_BM, 128), lambda i: (i, 0)),
    ]
    out_shape = [
        jax.ShapeDtypeStruct((N, n_out), jnp.float32),
        jax.ShapeDtypeStruct((N, 128), jnp.float32),
        jax.ShapeDtypeStruct((N, 128), jnp.float32),
    ]
  else:
    out_specs = [
        pl.BlockSpec((_BM, n_out), lambda i: (i, 0)),
        pl.BlockSpec((_BM, 128), lambda i: (i, 0)),
        pl.BlockSpec((_BM, 128), lambda i: (i, 0)),
    ]
    out_shape = [
        jax.ShapeDtypeStruct((N, n_out), jnp.float32),
        jax.ShapeDtypeStruct((N, 128), jnp.float32),
        jax.ShapeDtypeStruct((N, 128), jnp.float32),
    ]

  return pl.pallas_call(
      body,
      grid=(N // _BM,),
      in_specs=[
          pl.BlockSpec((_BM, H), lambda i: (i, 0)),
          pl.BlockSpec((NC, _BM, 128), lambda i: (0, i, 0)),
          pl.BlockSpec((NC, _BM, 128), lambda i: (0, i, 0)),
          pl.BlockSpec((1, H), lambda i: (0, 0)),
          pl.BlockSpec((H, n_out), lambda i: (0, 0)),
          pl.BlockSpec((H, n_out), lambda i: (0, 0)),
      ],
      out_specs=out_specs,
      out_shape=out_shape,
  )(s_prev, agg, deg, b, W_self, W_neigh)


def _tc_final(s3, agg3, deg, b):
  def body(sp_ref, a_ref, d_ref, b_ref, o_ref):
    d = d_ref[0, :, 0:1] + d_ref[1, :, 0:1]
    inv = 1.0 / jnp.maximum(d, 1.0)
    aggf = a_ref[0, :, :C] + a_ref[1, :, :C]
    o_ref[...] = sp_ref[...] + aggf * inv + b_ref[0]

  return pl.pallas_call(
      body,
      grid=(N // _BM,),
      in_specs=[
          pl.BlockSpec((_BM, C), lambda i: (i, 0)),
          pl.BlockSpec((NC, _BM, 128), lambda i: (0, i, 0)),
          pl.BlockSpec((NC, _BM, 128), lambda i: (0, i, 0)),
          pl.BlockSpec((1, C), lambda i: (0, 0)),
      ],
      out_specs=pl.BlockSpec((_BM, C), lambda i: (i, 0)),
      out_shape=jax.ShapeDtypeStruct((N, C), jnp.float32),
  )(s3, agg3, deg, b)


def kernel(x, edge_index, W_self1, W_neigh1, b1, W_self2, W_neigh2, b2,
           W_self3, W_neigh3, b3):
  src = edge_index[0].astype(jnp.int32)
  dst = edge_index[1].astype(jnp.int32)
  pad = EP - E
  src2d = jnp.concatenate(
      [src, jnp.zeros((pad,), jnp.int32)]).reshape(EROWS, CH)
  dst2d = jnp.concatenate(
      [dst, jnp.full((pad,), N, jnp.int32)]).reshape(EROWS, CH)

  zeros128 = jnp.zeros((ZL, 128), jnp.float32)
  ones128 = jnp.ones((CH, 128), jnp.float32)

  b1r = b1.reshape(1, H)
  b2r = b2.reshape(1, H)
  b3r = b3.reshape(1, C)

  s1, y10, y11 = _tc_first(x, W_self1, W_neigh1)
  agg1 = _seg_colsplit_nodeg(y10, y11, src2d, dst2d, zeros128)
  deg = _deg_edgesplit(dst2d, zeros128, ones128)
  s2, y20, y21 = _tc_mid(s1, agg1, deg, b1r, W_self2, W_neigh2, H, True)
  agg2 = _seg_colsplit_nodeg(y20, y21, src2d, dst2d, zeros128)
  s3, y3a, y3b = _tc_mid(s2, agg2, deg, b2r, W_self3, W_neigh3, C, False)
  agg3 = _seg_edgesplit(y3a, y3b, src2d, dst2d, zeros128)
  return _tc_final(s3, agg3, deg, b3r)
